# initial kernel scaffold (unmeasured)
import jax
import jax.numpy as jnp
from jax import lax
from jax.experimental import pallas as pl
from jax.experimental.pallas import tpu as pltpu


def kernel(
    x,
):
    def body(*refs):
        pass

    out_shape = jax.ShapeDtypeStruct(..., jnp.float32)
    return pl.pallas_call(body, out_shape=out_shape)(...)


# baseline (device time: 402993 ns/iter reference)
import jax
import jax.numpy as jnp
from jax import lax
from jax.experimental import pallas as pl
from jax.experimental.pallas import tpu as pltpu

N_DEV = 16


def kernel(x):
    m_per, n = x.shape

    def body(x_ref, out_ref, send_sems, recv_sems):
        my = lax.axis_index("i")
        left = (my - 1 + N_DEV) % N_DEV
        right = (my + 1) % N_DEV

        barrier_sem = pltpu.get_barrier_semaphore()
        for nbr in (left, right):
            pl.semaphore_signal(
                barrier_sem, inc=1,
                device_id=(nbr,), device_id_type=pl.DeviceIdType.MESH,
            )
        pl.semaphore_wait(barrier_sem, 2)

        out_ref[pl.ds(my * m_per, m_per), :] = x_ref[:, :].astype(jnp.bfloat16)

        for h in range(N_DEV - 1):
            origin = (my - h + N_DEV) % N_DEV
            rdma = pltpu.make_async_remote_copy(
                src_ref=out_ref.at[pl.ds(origin * m_per, m_per)],
                dst_ref=out_ref.at[pl.ds(origin * m_per, m_per)],
                send_sem=send_sems.at[h],
                recv_sem=recv_sems.at[h],
                device_id=(right,),
                device_id_type=pl.DeviceIdType.MESH,
            )
            rdma.start()
            rdma.wait()

    out_shape = jax.ShapeDtypeStruct((N_DEV * m_per, n), jnp.bfloat16)
    return pl.pallas_call(
        body,
        out_shape=out_shape,
        in_specs=[pl.BlockSpec(memory_space=pltpu.VMEM)],
        out_specs=pl.BlockSpec(memory_space=pltpu.VMEM),
        scratch_shapes=[
            pltpu.SemaphoreType.DMA((N_DEV - 1,)),
            pltpu.SemaphoreType.DMA((N_DEV - 1,)),
        ],
        compiler_params=pltpu.CompilerParams(collective_id=0),
    )(x)


# device time: 245110 ns/iter; 1.6441x vs baseline; 1.6441x over previous
import jax
import jax.numpy as jnp
from jax import lax
from jax.experimental import pallas as pl
from jax.experimental.pallas import tpu as pltpu

N_DEV = 16


def kernel(x):
    m_per, n = x.shape

    n_r = N_DEV // 2
    n_l = N_DEV - 1 - n_r

    def body(x_ref, out_ref, send_sems_r, recv_sems_r, send_sems_l, recv_sems_l):
        my = lax.axis_index("i")
        left = (my - 1 + N_DEV) % N_DEV
        right = (my + 1) % N_DEV

        barrier_sem = pltpu.get_barrier_semaphore()
        for nbr in (left, right):
            pl.semaphore_signal(
                barrier_sem, inc=1,
                device_id=(nbr,), device_id_type=pl.DeviceIdType.MESH,
            )
        pl.semaphore_wait(barrier_sem, 2)

        out_ref[pl.ds(my * m_per, m_per), :] = x_ref[:, :].astype(jnp.bfloat16)

        for h in range(n_r):
            origin_r = (my - h + N_DEV) % N_DEV
            rdma_r = pltpu.make_async_remote_copy(
                src_ref=out_ref.at[pl.ds(origin_r * m_per, m_per)],
                dst_ref=out_ref.at[pl.ds(origin_r * m_per, m_per)],
                send_sem=send_sems_r.at[h],
                recv_sem=recv_sems_r.at[h],
                device_id=(right,),
                device_id_type=pl.DeviceIdType.MESH,
            )
            rdma_r.start()
            if h < n_l:
                origin_l = (my + h) % N_DEV
                rdma_l = pltpu.make_async_remote_copy(
                    src_ref=out_ref.at[pl.ds(origin_l * m_per, m_per)],
                    dst_ref=out_ref.at[pl.ds(origin_l * m_per, m_per)],
                    send_sem=send_sems_l.at[h],
                    recv_sem=recv_sems_l.at[h],
                    device_id=(left,),
                    device_id_type=pl.DeviceIdType.MESH,
                )
                rdma_l.start()
                rdma_l.wait()
            rdma_r.wait()

    out_shape = jax.ShapeDtypeStruct((N_DEV * m_per, n), jnp.bfloat16)
    return pl.pallas_call(
        body,
        out_shape=out_shape,
        in_specs=[pl.BlockSpec(memory_space=pltpu.VMEM)],
        out_specs=pl.BlockSpec(memory_space=pltpu.VMEM),
        scratch_shapes=[
            pltpu.SemaphoreType.DMA((n_r,)),
            pltpu.SemaphoreType.DMA((n_r,)),
            pltpu.SemaphoreType.DMA((n_l,)),
            pltpu.SemaphoreType.DMA((n_l,)),
        ],
        compiler_params=pltpu.CompilerParams(collective_id=0),
    )(x)


# device time: 213088 ns/iter; 1.8912x vs baseline; 1.1503x over previous
import jax
import jax.numpy as jnp
from jax import lax
from jax.experimental import pallas as pl
from jax.experimental.pallas import tpu as pltpu

N_DEV = 16

N_HOPS = {("R", 0): 8, ("R", 1): 7, ("L", 0): 7, ("L", 1): 8}
KEYS = (("R", 0), ("R", 1), ("L", 0), ("L", 1))


def kernel(x):
    m_per, n = x.shape
    m_half = m_per // 2

    def body(x_ref, out_ref, *sems):
        sem_pairs = {k: (sems[2 * i], sems[2 * i + 1]) for i, k in enumerate(KEYS)}
        my = lax.axis_index("i")
        left = (my - 1 + N_DEV) % N_DEV
        right = (my + 1) % N_DEV

        barrier_sem = pltpu.get_barrier_semaphore()
        for nbr in (left, right):
            pl.semaphore_signal(
                barrier_sem, inc=1,
                device_id=(nbr,), device_id_type=pl.DeviceIdType.MESH,
            )
        pl.semaphore_wait(barrier_sem, 2)

        out_ref[pl.ds(my * m_per, m_per), :] = x_ref[:, :].astype(jnp.bfloat16)

        def make_rdma(key, hop):
            dirn, half = key
            origin = (my - hop + N_DEV) % N_DEV if dirn == "R" else (my + hop) % N_DEV
            rows = origin * m_per + half * m_half
            send_sems, recv_sems = sem_pairs[key]
            return pltpu.make_async_remote_copy(
                src_ref=out_ref.at[pl.ds(rows, m_half)],
                dst_ref=out_ref.at[pl.ds(rows, m_half)],
                send_sem=send_sems.at[hop],
                recv_sem=recv_sems.at[hop],
                device_id=(right if dirn == "R" else left,),
                device_id_type=pl.DeviceIdType.MESH,
            )

        descs = {}
        for key in KEYS:
            d = make_rdma(key, 0)
            d.start()
            descs[key + (0,)] = d

        for h in range(1, max(N_HOPS.values())):
            for key in KEYS:
                if h < N_HOPS[key]:
                    descs[key + (h - 1,)].wait_recv()
                    d = make_rdma(key, h)
                    d.start()
                    descs[key + (h,)] = d

        for key in KEYS:
            descs[key + (N_HOPS[key] - 1,)].wait_recv()
        for key in KEYS:
            for h in range(N_HOPS[key]):
                descs[key + (h,)].wait_send()

    out_shape = jax.ShapeDtypeStruct((N_DEV * m_per, n), jnp.bfloat16)
    scratch = []
    for key in KEYS:
        scratch.append(pltpu.SemaphoreType.DMA((N_HOPS[key],)))
        scratch.append(pltpu.SemaphoreType.DMA((N_HOPS[key],)))
    return pl.pallas_call(
        body,
        out_shape=out_shape,
        in_specs=[pl.BlockSpec(memory_space=pltpu.VMEM)],
        out_specs=pl.BlockSpec(memory_space=pltpu.VMEM),
        scratch_shapes=scratch,
        compiler_params=pltpu.CompilerParams(collective_id=0),
    )(x)


# device time: 211887 ns/iter; 1.9019x vs baseline; 1.0057x over previous
import jax
import jax.numpy as jnp
from jax import lax
from jax.experimental import pallas as pl
from jax.experimental.pallas import tpu as pltpu

N_DEV = 16

N_HOPS = {("R", 0): 8, ("R", 1): 7, ("L", 0): 7, ("L", 1): 8}
KEYS = (("R", 0), ("R", 1), ("L", 0), ("L", 1))
ROUND_ORDER = (("R", 0), ("L", 0), ("R", 1), ("L", 1))


def kernel(x):
    m_per, n = x.shape
    m_half = m_per // 2

    def body(x_ref, out_ref, *sems):
        sem_pairs = {k: (sems[2 * i], sems[2 * i + 1]) for i, k in enumerate(KEYS)}
        my = lax.axis_index("i")
        left = (my - 1 + N_DEV) % N_DEV
        right = (my + 1) % N_DEV

        barrier_sem = pltpu.get_barrier_semaphore()
        for nbr in (left, right):
            pl.semaphore_signal(
                barrier_sem, inc=1,
                device_id=(nbr,), device_id_type=pl.DeviceIdType.MESH,
            )
        pl.semaphore_wait(barrier_sem, 2)

        out_ref[pl.ds(my * m_per, m_per), :] = x_ref[:, :].astype(jnp.bfloat16)

        def make_rdma(key, hop):
            dirn, half = key
            origin = (my - hop + N_DEV) % N_DEV if dirn == "R" else (my + hop) % N_DEV
            rows = origin * m_per + half * m_half
            send_sems, recv_sems = sem_pairs[key]
            return pltpu.make_async_remote_copy(
                src_ref=out_ref.at[pl.ds(rows, m_half)],
                dst_ref=out_ref.at[pl.ds(rows, m_half)],
                send_sem=send_sems.at[hop],
                recv_sem=recv_sems.at[hop],
                device_id=(right if dirn == "R" else left,),
                device_id_type=pl.DeviceIdType.MESH,
            )

        descs = {}
        for key in KEYS:
            d = make_rdma(key, 0)
            d.start()
            descs[key + (0,)] = d

        for h in range(1, max(N_HOPS.values())):
            for key in ROUND_ORDER:
                if h < N_HOPS[key]:
                    descs[key + (h - 1,)].wait_recv()
                    d = make_rdma(key, h)
                    d.start()
                    descs[key + (h,)] = d

        for key in KEYS:
            descs[key + (N_HOPS[key] - 1,)].wait_recv()
        for key in KEYS:
            for h in range(N_HOPS[key]):
                descs[key + (h,)].wait_send()

    out_shape = jax.ShapeDtypeStruct((N_DEV * m_per, n), jnp.bfloat16)
    scratch = []
    for key in KEYS:
        scratch.append(pltpu.SemaphoreType.DMA((N_HOPS[key],)))
        scratch.append(pltpu.SemaphoreType.DMA((N_HOPS[key],)))
    return pl.pallas_call(
        body,
        out_shape=out_shape,
        in_specs=[pl.BlockSpec(memory_space=pltpu.VMEM)],
        out_specs=pl.BlockSpec(memory_space=pltpu.VMEM),
        scratch_shapes=scratch,
        compiler_params=pltpu.CompilerParams(collective_id=0),
    )(x)
